# baseline (speedup 1.0000x reference)
import functools

import jax
import jax.numpy as jnp
from jax import lax
from jax.experimental import pallas as pl
from jax.experimental.pallas import tpu as pltpu
from jax.experimental.pallas import tpu_sc as plsc

B = 16384
D = 32
NC = 2
NS = 16
NW = NC * NS
BPW = B // NW
L = 16
HD = 8
NB = BPW // L


def _wmf_body(uidx_hbm, iidx_hbm, utab_hbm, itab_hbm, out_hbm,
              idx_v, blk_sh, blk_v, urow_v, out_v,
              sem_a0, sem_a1, sem_b0, sem_b1):
    wid = lax.axis_index("s") * NC + lax.axis_index("c")
    sid = lax.axis_index("s")
    base = wid * BPW

    pltpu.sync_copy(uidx_hbm.at[wid], idx_v.at[0])
    pltpu.sync_copy(iidx_hbm.at[wid], idx_v.at[1])

    tabs = (utab_hbm, itab_hbm)
    sems_a = (sem_a0, sem_a1)
    sems_b = (sem_b0, sem_b1)

    def issue_a(t, a, b, slot):
        vec = idx_v[t, pl.ds(b * L, L)]
        arow = pl.multiple_of(HD * a, HD)
        for k in range(L):
            start = pl.multiple_of((vec[k] // 128) * 128, 128)
            pltpu.async_copy(
                tabs[t].at[pl.ds(arow, HD), pl.ds(start, 128)],
                blk_sh.at[sid, slot, k], sems_a[slot])

    def drain_a(t, slot):
        for k in range(L):
            pltpu.make_async_copy(
                tabs[t].at[pl.ds(0, HD), pl.ds(0, 128)],
                blk_sh.at[sid, slot, k], sems_a[slot]).wait()

    def issue_b(slot):
        pltpu.async_copy(blk_sh.at[sid, slot], blk_v.at[slot], sems_b[slot])

    def drain_b(slot):
        pltpu.make_async_copy(
            blk_sh.at[sid, slot], blk_v.at[slot], sems_b[slot]).wait()

    def extract(t, a, b, slot):
        vec = idx_v[t, pl.ds(b * L, L)]
        lane = lax.rem(vec, 128)
        k_v = lax.iota(jnp.int32, L)
        s_v = jnp.full((L,), slot, jnp.int32)
        sl = pl.ds(b * L, L)
        if t == 1:
            acc = jnp.zeros((L,), jnp.float32)
        for dl in range(HD):
            d_v = jnp.full((L,), dl, jnp.int32)
            val = plsc.load_gather(blk_v, [s_v, k_v, d_v, lane])
            if t == 0:
                urow_v[HD * a + dl, sl] = val
            else:
                acc = acc + urow_v[HD * a + dl, sl] * val
        if t == 1:
            out_v[sl] = out_v[sl] + acc

    zero = jnp.zeros((L,), jnp.float32)
    for b in range(NB):
        out_v[pl.ds(b * L, L)] = zero

    for t in range(2):

        def a_pass(a, acarry, t=t):
            issue_a(t, a, 0, 0)
            issue_a(t, a, 1, 1)
            drain_a(t, 0)
            issue_b(0)

            def body(h, carry, t=t, a=a):
                b0 = 2 * h
                drain_b(0)
                extract(t, a, b0, 0)

                @pl.when(b0 + 2 < NB)
                def _():
                    issue_a(t, a, b0 + 2, 0)

                drain_a(t, 1)
                issue_b(1)

                drain_b(1)
                extract(t, a, b0 + 1, 1)

                @pl.when(b0 + 3 < NB)
                def _():
                    issue_a(t, a, b0 + 3, 1)

                @pl.when(b0 + 2 < NB)
                def _():
                    drain_a(t, 0)
                    issue_b(0)

                return carry

            lax.fori_loop(0, NB // 2, body, 0)
            return acarry

        lax.fori_loop(0, D // HD, a_pass, 0)

    pltpu.sync_copy(out_v, out_hbm.at[pl.ds(base, BPW)])


@jax.jit
def kernel(user_input, item_input, user_table, item_table):
    uidx = user_input.astype(jnp.int32).reshape(NW, BPW)
    iidx = item_input.astype(jnp.int32).reshape(NW, BPW)
    mesh = plsc.VectorSubcoreMesh(core_axis_name="c", subcore_axis_name="s")
    f = functools.partial(
        pl.kernel,
        mesh=mesh,
        out_type=jax.ShapeDtypeStruct((B,), jnp.float32),
        scratch_types=[
            pltpu.VMEM((2, BPW), jnp.int32),
            pltpu.VMEM_SHARED((NS, 2, L, HD, 128), jnp.float32),
            pltpu.VMEM((2, L, HD, 128), jnp.float32),
            pltpu.VMEM((D, BPW), jnp.float32),
            pltpu.VMEM((BPW,), jnp.float32),
            pltpu.SemaphoreType.DMA,
            pltpu.SemaphoreType.DMA,
            pltpu.SemaphoreType.DMA,
            pltpu.SemaphoreType.DMA,
        ],
        compiler_params=pltpu.CompilerParams(
            needs_layout_passes=False, use_tc_tiling_on_sc=True),
    )(_wmf_body)
    return f(uidx, iidx, user_table.T, item_table.T)


# speedup vs baseline: 1.0708x; 1.0708x over previous
import functools

import jax
import jax.numpy as jnp
from jax import lax
from jax.experimental import pallas as pl
from jax.experimental.pallas import tpu as pltpu
from jax.experimental.pallas import tpu_sc as plsc

B = 16384
D = 32
NC = 2
NS = 16
NW = NC * NS
BPW = B // NW
L = 16
KB = 8
NPAIR = BPW // (2 * KB)
HD = 8
NB = BPW // L


def _wmf_body(uidx_hbm, iidx_hbm, utab_hbm, itab_hbm, out_hbm,
              idx_v, blkd_v, blks_sh, blks_v, urow_v, out_v,
              semd0, semd1, sema0, sema1, semb0, semb1):
    wid = lax.axis_index("s") * NC + lax.axis_index("c")
    sid = lax.axis_index("s")
    base = wid * BPW

    pltpu.sync_copy(uidx_hbm.at[wid], idx_v.at[0])
    pltpu.sync_copy(iidx_hbm.at[wid], idx_v.at[1])

    tabs = (utab_hbm, itab_hbm)

    sems_d = (semd0, semd1)

    def issue_d(t, a, b, slot):
        vec = idx_v[t, pl.ds(b * L, L)]
        arow = pl.multiple_of(HD * a, HD)
        for k in range(L):
            start = pl.multiple_of((vec[k] // 128) * 128, 128)
            pltpu.async_copy(
                tabs[t].at[pl.ds(arow, HD), pl.ds(start, 128)],
                blkd_v.at[slot, k], sems_d[slot])

    def drain_d(t, slot):
        for k in range(L):
            pltpu.make_async_copy(
                tabs[t].at[pl.ds(0, HD), pl.ds(0, 128)],
                blkd_v.at[slot, k], sems_d[slot]).wait()

    def extract_d(t, a, b, slot):
        vec = idx_v[t, pl.ds(b * L, L)]
        lane = lax.rem(vec, 128)
        k_v = lax.iota(jnp.int32, L)
        s_v = jnp.full((L,), slot, jnp.int32)
        sl = pl.ds(b * L, L)
        if t == 1:
            acc = jnp.zeros((L,), jnp.float32)
        for dl in range(HD):
            d_v = jnp.full((L,), dl, jnp.int32)
            val = plsc.load_gather(blkd_v, [s_v, k_v, d_v, lane])
            if t == 0:
                urow_v[HD * a + dl, sl] = val
            else:
                acc = acc + urow_v[HD * a + dl, sl] * val
        if t == 1:
            out_v[sl] = out_v[sl] + acc

    def direct_path():
        zero = jnp.zeros((L,), jnp.float32)
        for b in range(NB):
            out_v[pl.ds(b * L, L)] = zero

        for t in range(2):

            def a_pass(a, acarry, t=t):
                issue_d(t, a, 0, 0)
                issue_d(t, a, 1, 1)

                def body(h, carry, t=t, a=a):
                    b0 = 2 * h
                    drain_d(t, 0)
                    extract_d(t, a, b0, 0)

                    @pl.when(b0 + 2 < NB)
                    def _():
                        issue_d(t, a, b0 + 2, 0)

                    drain_d(t, 1)
                    extract_d(t, a, b0 + 1, 1)

                    @pl.when(b0 + 3 < NB)
                    def _():
                        issue_d(t, a, b0 + 3, 1)

                    return carry

                lax.fori_loop(0, NB // 2, body, 0)
                return acarry

            lax.fori_loop(0, D // HD, a_pass, 0)

    sems_a = (sema0, sema1)
    sems_b = (semb0, semb1)
    ssid = sid // 2

    def issue_a(t, a, b, slot):
        vec = idx_v[t, pl.ds(b * L, L)]
        arow = pl.multiple_of(HD * a, HD)
        for k in range(L):
            start = pl.multiple_of((vec[k] // 128) * 128, 128)
            pltpu.async_copy(
                tabs[t].at[pl.ds(arow, HD), pl.ds(start, 128)],
                blks_sh.at[ssid, slot, k], sems_a[slot])

    def drain_a(t, slot):
        for k in range(L):
            pltpu.make_async_copy(
                tabs[t].at[pl.ds(0, HD), pl.ds(0, 128)],
                blks_sh.at[ssid, slot, k], sems_a[slot]).wait()

    def issue_b(slot):
        pltpu.async_copy(blks_sh.at[ssid, slot], blks_v.at[slot],
                         sems_b[slot])

    def drain_b(slot):
        pltpu.make_async_copy(
            blks_sh.at[ssid, slot], blks_v.at[slot], sems_b[slot]).wait()

    def extract_s(t, a, b, slot):
        vec = idx_v[t, pl.ds(b * L, L)]
        lane = lax.rem(vec, 128)
        k_v = lax.iota(jnp.int32, L)
        s_v = jnp.full((L,), slot, jnp.int32)
        sl = pl.ds(b * L, L)
        if t == 1:
            acc = jnp.zeros((L,), jnp.float32)
        for dl in range(HD):
            d_v = jnp.full((L,), dl, jnp.int32)
            val = plsc.load_gather(blks_v, [s_v, k_v, d_v, lane])
            if t == 0:
                urow_v[HD * a + dl, sl] = val
            else:
                acc = acc + urow_v[HD * a + dl, sl] * val
        if t == 1:
            out_v[sl] = out_v[sl] + acc

    def spmem_path():
        zero = jnp.zeros((L,), jnp.float32)
        for b in range(NB):
            out_v[pl.ds(b * L, L)] = zero

        for t in range(2):

            def a_pass(a, acarry, t=t):
                issue_a(t, a, 0, 0)
                issue_a(t, a, 1, 1)
                drain_a(t, 0)
                issue_b(0)

                def body(h, carry, t=t, a=a):
                    b0 = 2 * h
                    drain_b(0)
                    extract_s(t, a, b0, 0)

                    @pl.when(b0 + 2 < NB)
                    def _():
                        issue_a(t, a, b0 + 2, 0)

                    drain_a(t, 1)
                    issue_b(1)

                    drain_b(1)
                    extract_s(t, a, b0 + 1, 1)

                    @pl.when(b0 + 3 < NB)
                    def _():
                        issue_a(t, a, b0 + 3, 1)

                    @pl.when(b0 + 2 < NB)
                    def _():
                        drain_a(t, 0)
                        issue_b(0)

                    return carry

                lax.fori_loop(0, NB // 2, body, 0)
                return acarry

            lax.fori_loop(0, D // HD, a_pass, 0)

    @pl.when(lax.rem(sid, 2) == 0)
    def _():
        direct_path()

    @pl.when(lax.rem(sid, 2) == 1)
    def _():
        spmem_path()

    pltpu.sync_copy(out_v, out_hbm.at[pl.ds(base, BPW)])


@jax.jit
def kernel(user_input, item_input, user_table, item_table):
    uidx = user_input.astype(jnp.int32).reshape(NW, BPW)
    iidx = item_input.astype(jnp.int32).reshape(NW, BPW)
    mesh = plsc.VectorSubcoreMesh(core_axis_name="c", subcore_axis_name="s")
    f = functools.partial(
        pl.kernel,
        mesh=mesh,
        out_type=jax.ShapeDtypeStruct((B,), jnp.float32),
        scratch_types=[
            pltpu.VMEM((2, BPW), jnp.int32),
            pltpu.VMEM((2, L, HD, 128), jnp.float32),
            pltpu.VMEM_SHARED((NS // 2, 2, L, HD, 128), jnp.float32),
            pltpu.VMEM((2, L, HD, 128), jnp.float32),
            pltpu.VMEM((D, BPW), jnp.float32),
            pltpu.VMEM((BPW,), jnp.float32),
            pltpu.SemaphoreType.DMA,
            pltpu.SemaphoreType.DMA,
            pltpu.SemaphoreType.DMA,
            pltpu.SemaphoreType.DMA,
            pltpu.SemaphoreType.DMA,
            pltpu.SemaphoreType.DMA,
        ],
        compiler_params=pltpu.CompilerParams(
            needs_layout_passes=False, use_tc_tiling_on_sc=True),
    )(_wmf_body)
    return f(uidx, iidx, user_table.T, item_table.T)


# speedup vs baseline: 1.2992x; 1.2133x over previous
import functools

import jax
import jax.numpy as jnp
from jax import lax
from jax.experimental import pallas as pl
from jax.experimental.pallas import tpu as pltpu
from jax.experimental.pallas import tpu_sc as plsc

B = 16384
D = 32
NC = 2
NS = 16
NW = NC * NS
BPW = B // NW
L = 16
KB = 8
NPAIR = BPW // (2 * KB)


def _wmf_body(uidx_hbm, iidx_hbm, utab_hbm, itab_hbm, out_hbm,
              idx_v, blk_v, urow_v, out_v, sem0, sem1):
    wid = lax.axis_index("s") * NC + lax.axis_index("c")
    base = wid * BPW

    pltpu.sync_copy(uidx_hbm.at[wid], idx_v.at[0])
    pltpu.sync_copy(iidx_hbm.at[wid], idx_v.at[1])

    tabs = (utab_hbm, itab_hbm)
    sems = (sem0, sem1)

    def issue_pair(t, g):
        vec = idx_v[t, pl.ds(g * 2 * KB, L)]
        for k in range(L):
            start = pl.multiple_of((vec[k] // 128) * 128, 128)
            pltpu.async_copy(
                tabs[t].at[:, pl.ds(start, 128)],
                blk_v.at[k // KB, k % KB], sems[k // KB])

    def drain(t, slot):
        for k in range(KB):
            pltpu.make_async_copy(
                tabs[t].at[:, pl.ds(0, 128)],
                blk_v.at[slot, k], sems[slot]).wait()

    def extract_pair(t, g):
        vec = idx_v[t, pl.ds(g * 2 * KB, L)]
        lane = lax.rem(vec, 128)
        j = lax.iota(jnp.int32, L)
        s_v = j // KB
        k_v = lax.rem(j, KB)
        sl = pl.ds(g * 2 * KB, L)
        if t == 1:
            acc = jnp.zeros((L,), jnp.float32)
        for d in range(D):
            d_v = jnp.full((L,), d, jnp.int32)
            val = plsc.load_gather(blk_v, [s_v, k_v, d_v, lane])
            if t == 0:
                urow_v[d, sl] = val
            else:
                acc = acc + urow_v[d, sl] * val
        if t == 1:
            out_v[sl] = acc

    for t in range(2):
        issue_pair(t, 0)

        def body(g, carry, t=t):
            drain(t, 0)
            drain(t, 1)
            extract_pair(t, g)

            @pl.when(g < NPAIR - 1)
            def _():
                issue_pair(t, g + 1)

            return carry

        lax.fori_loop(0, NPAIR, body, 0)

    pltpu.sync_copy(out_v, out_hbm.at[pl.ds(base, BPW)])


@jax.jit
def kernel(user_input, item_input, user_table, item_table):
    uidx = user_input.astype(jnp.int32).reshape(NW, BPW)
    iidx = item_input.astype(jnp.int32).reshape(NW, BPW)
    mesh = plsc.VectorSubcoreMesh(core_axis_name="c", subcore_axis_name="s")
    f = functools.partial(
        pl.kernel,
        mesh=mesh,
        out_type=jax.ShapeDtypeStruct((B,), jnp.float32),
        scratch_types=[
            pltpu.VMEM((2, BPW), jnp.int32),
            pltpu.VMEM((2, KB, D, 128), jnp.float32),
            pltpu.VMEM((D, BPW), jnp.float32),
            pltpu.VMEM((BPW,), jnp.float32),
            pltpu.SemaphoreType.DMA,
            pltpu.SemaphoreType.DMA,
        ],
        compiler_params=pltpu.CompilerParams(
            needs_layout_passes=False, use_tc_tiling_on_sc=True),
    )(_wmf_body)
    return f(uidx, iidx, user_table.T, item_table.T)
